# baseline (speedup 1.0000x reference)
import functools

import jax
import jax.numpy as jnp
from jax import lax
from jax.experimental import pallas as pl
from jax.experimental.pallas import tpu as pltpu
from jax.experimental.pallas import tpu_sc as plsc

D = 64
L = 16
NC = 2
NS = 16
NW = NC * NS
C = 128
EPS = 1e-12


_SHUF_DN = lax.GatherDimensionNumbers(
    offset_dims=(), collapsed_slice_dims=(0,), start_index_map=(0,))


def _lane_shuffle(x, p):
    return lax.gather(x, p[:, None], _SHUF_DN, (1,),
                      mode=lax.GatherScatterMode.PROMISE_IN_BOUNDS)


def _rsqrt(v):
    i = plsc.bitcast(v, jnp.int32)
    i = jnp.int32(0x5F3759DF) - lax.shift_right_logical(i, 1)
    y = plsc.bitcast(i, jnp.float32)
    half = v * 0.5
    for _ in range(3):
        y = y * (1.5 - half * y * y)
    return y


def _body(seq, nch, ids, tok, pos, gam, bet, out,
          pos_v, g_v, b_v, idx_v, row_v, out_v, sem):
    cid = lax.axis_index("c")
    sid = lax.axis_index("s")
    wid = sid * NC + cid
    per_w = nch * C

    pltpu.sync_copy(pos.at[pl.ds(0, seq)], pos_v)
    pltpu.sync_copy(gam, g_v)
    pltpu.sync_copy(bet, b_v)

    gs = [g_v[pl.ds(16 * k, 16)] for k in range(4)]
    bs = [b_v[pl.ds(16 * k, 16)] for k in range(4)]
    perms = [jnp.bitwise_xor(lax.iota(jnp.int32, 16), jnp.int32(sh))
             for sh in (8, 4, 2, 1)]

    base_w = wid * per_w
    s0_init = lax.rem(base_w, seq)

    def chunk(g, s0):
        base = base_w + g * C
        pltpu.sync_copy(ids.at[pl.ds(base, C)], idx_v)
        pltpu.async_copy(tok.at[idx_v], row_v, sem).wait()

        def rowfn(i, s):
            xs = [row_v[i, pl.ds(16 * k, 16)] + pos_v[s, pl.ds(16 * k, 16)]
                  for k in range(4)]
            s_v = (xs[0] + xs[1]) + (xs[2] + xs[3])
            q_v = (xs[0] * xs[0] + xs[1] * xs[1]) + (xs[2] * xs[2] + xs[3] * xs[3])
            for p in perms:
                s_v = s_v + _lane_shuffle(s_v, p)
                q_v = q_v + _lane_shuffle(q_v, p)
            mean = s_v * (1.0 / D)
            var = q_v * (1.0 / D) - mean * mean
            r = _rsqrt(var + EPS)
            for k in range(4):
                out_v[i, pl.ds(16 * k, 16)] = (xs[k] - mean) * r * gs[k] + bs[k]
            s = s + 1
            return jnp.where(s == seq, 0, s)

        s0 = lax.fori_loop(0, C, rowfn, s0)
        pltpu.sync_copy(out_v, out.at[pl.ds(base, C)])
        return s0

    lax.fori_loop(0, nch, chunk, s0_init)


@functools.lru_cache(maxsize=None)
def _build(nrows, seq):
    assert nrows % (NW * C) == 0
    nch = nrows // (NW * C)
    mesh = plsc.VectorSubcoreMesh(core_axis_name="c", subcore_axis_name="s")
    return pl.kernel(
        functools.partial(_body, seq, nch),
        out_type=jax.ShapeDtypeStruct((nrows, D), jnp.float32),
        mesh=mesh,
        compiler_params=pltpu.CompilerParams(
            needs_layout_passes=False, use_tc_tiling_on_sc=False),
        scratch_types=[
            pltpu.VMEM((seq, D), jnp.float32),
            pltpu.VMEM((D,), jnp.float32),
            pltpu.VMEM((D,), jnp.float32),
            pltpu.VMEM((C,), jnp.int32),
            pltpu.VMEM((C, D), jnp.float32),
            pltpu.VMEM((C, D), jnp.float32),
            pltpu.SemaphoreType.DMA,
        ],
    )


def kernel(input_ids, token_table, pos_table, gamma, beta):
    batch, seq = input_ids.shape
    ids_flat = input_ids.reshape(-1).astype(jnp.int32)
    out_flat = _build(batch * seq, seq)(
        ids_flat, token_table, pos_table, gamma, beta)
    return out_flat.reshape(batch, seq, D)
